# baseline (speedup 1.0000x reference)
import jax
import jax.numpy as jnp
from jax import lax
from jax.experimental import pallas as pl
from jax.experimental.pallas import tpu as pltpu
from jax.experimental.pallas import tpu_sc as plsc

NC = 2
NS = 16
NW = NC * NS
CH = 128


def _dis_tc(deg2, np_pad):
    def body(d_ref, dis_ref, sw_ref):
        d = d_ref[...]
        dsum = d[:np_pad, 0] + 1.0
        dis = lax.rsqrt(jnp.maximum(dsum, 1e-12))
        dis_ref[...] = dis.reshape(1, np_pad)
        sw_ref[...] = (dis * dis).reshape(1, np_pad)

    return pl.pallas_call(
        body,
        out_shape=[
            jax.ShapeDtypeStruct((1, np_pad), jnp.float32),
            jax.ShapeDtypeStruct((1, np_pad), jnp.float32),
        ],
    )(deg2)


def _make_norm(np_pad, nre, kch):
    def body(src_hbm, dstr_hbm, wr_hbm, dis_hbm, norm_hbm,
             disv, srcv, dstnv, wv, normv):
        c = lax.axis_index("c")
        s = lax.axis_index("s")
        wid = c * NS + s
        r0 = wid * kch

        pltpu.sync_copy(dis_hbm, disv)
        pltpu.sync_copy(src_hbm.at[pl.ds(r0, kch)], srcv)
        pltpu.sync_copy(dstr_hbm.at[pl.ds(r0, kch)], dstnv)
        pltpu.sync_copy(wr_hbm.at[pl.ds(r0, kch)], wv)
        for j in range(kch):
            for k in range(CH // 16):
                sl = pl.ds(k * 16, 16)
                nrm = (plsc.load_gather(disv, [srcv[j, sl]]) * wv[j, sl]
                       * plsc.load_gather(disv, [dstnv[j, sl]]))
                normv[pl.ds(j * CH + k * 16, 16)] = nrm
        pltpu.sync_copy(normv, norm_hbm.at[pl.ds(r0 * CH, kch * CH)])

    mesh = plsc.VectorSubcoreMesh(core_axis_name="c", subcore_axis_name="s",
                                  num_cores=NC, num_subcores=NS)
    return pl.kernel(
        body,
        out_type=jax.ShapeDtypeStruct((nre * CH,), jnp.float32),
        mesh=mesh,
        compiler_params=pltpu.CompilerParams(needs_layout_passes=False),
        scratch_types=[
            pltpu.VMEM((np_pad,), jnp.float32),
            pltpu.VMEM((kch, CH), jnp.int32),
            pltpu.VMEM((kch, CH), jnp.int32),
            pltpu.VMEM((kch, CH), jnp.float32),
            pltpu.VMEM((kch * CH,), jnp.float32),
        ],
    )


def _make_layer(n_nodes, nrows, cw, dh):
    pt = n_nodes // NS
    kpt = nrows // NS
    npass = 2
    kh = kpt // npass
    ng = kh // 2

    def body(g_hbm, gi_hbm, src_hbm, dst_hbm, nrm_hbm, agg_hbm,
             acc, srcv, dstv, normv, rowbuf, sem0, sem1):
        c = lax.axis_index("c")
        s = lax.axis_index("s")
        nbase = c * n_nodes + s * pt

        pltpu.sync_copy(gi_hbm.at[pl.ds(nbase, pt)], acc.at[pl.ds(s * pt, pt)])
        plsc.subcore_barrier()

        sems = [sem0, sem1]

        def issue(j, b):
            return pltpu.async_copy(g_hbm.at[srcv.at[j]], rowbuf.at[b],
                                    sems[b])

        def wait(j, b):
            pltpu.make_async_copy(g_hbm.at[srcv.at[j]], rowbuf.at[b],
                                  sems[b]).wait()

        def scale_scatter(j, b):
            jbase = jnp.zeros((16,), jnp.int32) + j * cw

            def row_body(r, carry):
                nv = plsc.load_gather(normv, [jbase + r])
                for t in range(dh // 16):
                    sl = (b, r, pl.ds(t * 16, 16))
                    rowbuf[sl] = rowbuf[sl] * nv
                return carry

            lax.fori_loop(0, cw, row_body, 0)
            pltpu.sync_copy(rowbuf.at[b], acc.at[dstv.at[j]], add=True)

        for half in range(npass):
            r0 = s * kpt + half * kh
            pltpu.sync_copy(src_hbm.at[pl.ds(c * nrows + r0, kh)], srcv)
            pltpu.sync_copy(dst_hbm.at[pl.ds(r0, kh)], dstv)
            pltpu.sync_copy(nrm_hbm.at[pl.ds(r0 * cw, kh * cw)], normv)
            issue(0, 0)

            def chunk_body(g, carry):
                issue(2 * g + 1, 1)
                wait(2 * g, 0)
                scale_scatter(2 * g, 0)

                @pl.when(g < ng - 1)
                def _():
                    issue(2 * g + 2, 0)

                wait(2 * g + 1, 1)
                scale_scatter(2 * g + 1, 1)
                return carry

            lax.fori_loop(0, ng, chunk_body, 0)

        plsc.subcore_barrier()
        pltpu.sync_copy(acc.at[pl.ds(s * pt, pt)], agg_hbm.at[pl.ds(nbase, pt)])

    mesh = plsc.VectorSubcoreMesh(core_axis_name="c", subcore_axis_name="s",
                                  num_cores=NC, num_subcores=NS)
    return pl.kernel(
        body,
        out_type=jax.ShapeDtypeStruct((NC * n_nodes, dh), jnp.float32),
        mesh=mesh,
        compiler_params=pltpu.CompilerParams(needs_layout_passes=False),
        scratch_types=[
            pltpu.VMEM_SHARED((n_nodes, dh), jnp.float32),
            pltpu.VMEM((kpt // npass, cw), jnp.int32),
            pltpu.VMEM((kpt // npass, cw), jnp.int32),
            pltpu.VMEM((kpt // npass * cw,), jnp.float32),
            pltpu.VMEM((2, cw, dh), jnp.float32),
            pltpu.SemaphoreType.DMA,
            pltpu.SemaphoreType.DMA,
        ],
    )


def _mm_first(x, w, sw, *, bm, dh):
    n, d = x.shape

    def body(x_ref, w_ref, sw_ref, g_ref, gi_ref):
        g = jnp.dot(x_ref[...], w_ref[...], preferred_element_type=jnp.float32)
        gi = g * sw_ref[...]
        g_ref[0] = g[:, :dh]
        g_ref[1] = g[:, dh:]
        gi_ref[0] = gi[:, :dh]
        gi_ref[1] = gi[:, dh:]

    return pl.pallas_call(
        body,
        grid=(n // bm,),
        in_specs=[
            pl.BlockSpec((bm, d), lambda i: (i, 0)),
            pl.BlockSpec((d, d), lambda i: (0, 0)),
            pl.BlockSpec((bm, 1), lambda i: (i, 0)),
        ],
        out_specs=[
            pl.BlockSpec((NC, bm, dh), lambda i: (0, i, 0)),
            pl.BlockSpec((NC, bm, dh), lambda i: (0, i, 0)),
        ],
        out_shape=[
            jax.ShapeDtypeStruct((NC, n, dh), jnp.float32),
            jax.ShapeDtypeStruct((NC, n, dh), jnp.float32),
        ],
    )(x, w, sw)


def _mm_mid(agg, b_prev, w, sw, *, bm, dh):
    nc, n, _ = agg.shape
    d = w.shape[0]

    def body(a_ref, b_ref, w_ref, sw_ref, g_ref, gi_ref):
        h = jnp.concatenate([a_ref[0], a_ref[1]], axis=1) + b_ref[...]
        h = jnp.maximum(h, 0.0)
        g = jnp.dot(h, w_ref[...], preferred_element_type=jnp.float32)
        gi = g * sw_ref[...]
        g_ref[0] = g[:, :dh]
        g_ref[1] = g[:, dh:]
        gi_ref[0] = gi[:, :dh]
        gi_ref[1] = gi[:, dh:]

    return pl.pallas_call(
        body,
        grid=(n // bm,),
        in_specs=[
            pl.BlockSpec((NC, bm, dh), lambda i: (0, i, 0)),
            pl.BlockSpec((1, d), lambda i: (0, 0)),
            pl.BlockSpec((d, d), lambda i: (0, 0)),
            pl.BlockSpec((bm, 1), lambda i: (i, 0)),
        ],
        out_specs=[
            pl.BlockSpec((NC, bm, dh), lambda i: (0, i, 0)),
            pl.BlockSpec((NC, bm, dh), lambda i: (0, i, 0)),
        ],
        out_shape=[
            jax.ShapeDtypeStruct((NC, n, dh), jnp.float32),
            jax.ShapeDtypeStruct((NC, n, dh), jnp.float32),
        ],
    )(agg, b_prev, w, sw)


def _mm_last(agg, b_prev, w, b_out, *, bm, dh):
    nc, n, _ = agg.shape
    d = w.shape[0]

    def body(a_ref, b_ref, w_ref, bo_ref, o_ref):
        h = jnp.concatenate([a_ref[0], a_ref[1]], axis=1) + b_ref[...]
        h = jnp.maximum(h, 0.0)
        o_ref[...] = (jnp.dot(h, w_ref[...], preferred_element_type=jnp.float32)
                      + bo_ref[...])

    return pl.pallas_call(
        body,
        grid=(n // bm,),
        in_specs=[
            pl.BlockSpec((NC, bm, dh), lambda i: (0, i, 0)),
            pl.BlockSpec((1, d), lambda i: (0, 0)),
            pl.BlockSpec((d, d), lambda i: (0, 0)),
            pl.BlockSpec((1, d), lambda i: (0, 0)),
        ],
        out_specs=pl.BlockSpec((bm, d), lambda i: (i, 0)),
        out_shape=jax.ShapeDtypeStruct((n, d), jnp.float32),
    )(agg, b_prev, w, b_out)


def kernel(x, edge_index, edge_attr, W1, b1, W2, b2, W3, b3, Wout, bout):
    n, d = x.shape
    e = edge_index.shape[1]
    dh = d // 2
    cw = 64
    kch = -(-e // (NW * CH))
    ep = NW * CH * kch
    nre = ep // CH
    nr2 = ep // cw
    npad = -(-n // 512) * 512
    bm = 512

    pad = ep - e
    src = jnp.concatenate([edge_index[0], jnp.zeros((pad,), jnp.int32)])
    dst = jnp.concatenate([edge_index[1], jnp.zeros((pad,), jnp.int32)])
    wp = jnp.concatenate([edge_attr, jnp.zeros((pad,), edge_attr.dtype)])
    src2 = jnp.stack([src, src + npad]).reshape(NC * nr2, cw)
    srcn = src.reshape(nre, CH)
    dstn = dst.reshape(nre, CH)
    dst2 = dst.reshape(nr2, cw)
    wr = wp.reshape(nre, CH)
    xp = jnp.pad(x, ((0, npad - n), (0, 0)))

    layer = _make_layer(npad, nr2, cw, dh)

    ones_t = jnp.ones((8, dh), jnp.float32)
    zeros_g = jnp.zeros((NC * npad, dh), jnp.float32)
    zidx = jnp.zeros((NC * nr2, cw), jnp.int32)
    deg2 = layer(ones_t, zeros_g, zidx, dst2, wp)
    dis, selfw = _dis_tc(deg2, npad)
    norm_r = _make_norm(npad, nre, kch)(srcn, dstn, wr, dis.reshape(npad))
    sw2 = selfw.reshape(npad, 1)

    g, gi = _mm_first(xp, W1, sw2, bm=bm, dh=dh)
    agg = layer(g.reshape(NC * npad, dh), gi.reshape(NC * npad, dh),
                src2, dst2, norm_r).reshape(NC, npad, dh)
    g, gi = _mm_mid(agg, b1.reshape(1, d), W2, sw2, bm=bm, dh=dh)
    agg = layer(g.reshape(NC * npad, dh), gi.reshape(NC * npad, dh),
                src2, dst2, norm_r).reshape(NC, npad, dh)
    g, gi = _mm_mid(agg, b2.reshape(1, d), W3, sw2, bm=bm, dh=dh)
    agg = layer(g.reshape(NC * npad, dh), gi.reshape(NC * npad, dh),
                src2, dst2, norm_r).reshape(NC, npad, dh)
    out = _mm_last(agg, b3.reshape(1, d), Wout, bout.reshape(1, d),
                   bm=bm, dh=dh)
    return out[:n]


# speedup vs baseline: 8.3669x; 8.3669x over previous
import jax
import jax.numpy as jnp
from jax import lax
from jax.experimental import pallas as pl
from jax.experimental.pallas import tpu as pltpu
from jax.experimental.pallas import tpu_sc as plsc

NC = 2
NS = 16
NW = NC * NS
CH = 128


def _dis_tc(deg2, np_pad):
    def body(d_ref, dis_ref, sw_ref):
        d = d_ref[...]
        dsum = d[:np_pad, 0] + 1.0
        dis = lax.rsqrt(jnp.maximum(dsum, 1e-12))
        dis_ref[...] = dis.reshape(1, np_pad)
        sw_ref[...] = (dis * dis).reshape(1, np_pad)

    return pl.pallas_call(
        body,
        out_shape=[
            jax.ShapeDtypeStruct((1, np_pad), jnp.float32),
            jax.ShapeDtypeStruct((1, np_pad), jnp.float32),
        ],
    )(deg2)


def _make_norm(np_pad, nre, kch):
    def body(src_hbm, dstr_hbm, wr_hbm, dis_hbm, norm_hbm,
             disv, srcv, dstnv, wv, normv):
        c = lax.axis_index("c")
        s = lax.axis_index("s")
        wid = c * NS + s
        r0 = wid * kch

        pltpu.sync_copy(dis_hbm, disv)
        pltpu.sync_copy(src_hbm.at[pl.ds(r0, kch)], srcv)
        pltpu.sync_copy(dstr_hbm.at[pl.ds(r0, kch)], dstnv)
        pltpu.sync_copy(wr_hbm.at[pl.ds(r0, kch)], wv)
        for j in range(kch):
            for k in range(CH // 16):
                sl = pl.ds(k * 16, 16)
                nrm = (plsc.load_gather(disv, [srcv[j, sl]]) * wv[j, sl]
                       * plsc.load_gather(disv, [dstnv[j, sl]]))
                normv[pl.ds(j * CH + k * 16, 16)] = nrm
        pltpu.sync_copy(normv, norm_hbm.at[pl.ds(r0 * CH, kch * CH)])

    mesh = plsc.VectorSubcoreMesh(core_axis_name="c", subcore_axis_name="s",
                                  num_cores=NC, num_subcores=NS)
    return pl.kernel(
        body,
        out_type=jax.ShapeDtypeStruct((nre * CH,), jnp.float32),
        mesh=mesh,
        compiler_params=pltpu.CompilerParams(needs_layout_passes=False),
        scratch_types=[
            pltpu.VMEM((np_pad,), jnp.float32),
            pltpu.VMEM((kch, CH), jnp.int32),
            pltpu.VMEM((kch, CH), jnp.int32),
            pltpu.VMEM((kch, CH), jnp.float32),
            pltpu.VMEM((kch * CH,), jnp.float32),
        ],
    )


def _make_layer(n_nodes, nrows, cw, dh):
    pt = n_nodes // NS
    kpt = nrows // NS
    npass = 2
    kh = kpt // npass
    ng = kh // 2

    def body(g_hbm, gi_hbm, src_hbm, dst_hbm, nrm_hbm, agg_hbm,
             acc, srcv, dstv, normv, rowbuf, sem0, sem1):
        c = lax.axis_index("c")
        s = lax.axis_index("s")
        nbase = c * n_nodes + s * pt

        pltpu.sync_copy(gi_hbm.at[pl.ds(nbase, pt)], acc.at[pl.ds(s * pt, pt)])
        plsc.subcore_barrier()

        sems = [sem0, sem1]

        def issue(j, b):
            return pltpu.async_copy(g_hbm.at[srcv.at[j]], rowbuf.at[b],
                                    sems[b])

        def wait(j, b):
            pltpu.make_async_copy(g_hbm.at[srcv.at[j]], rowbuf.at[b],
                                  sems[b]).wait()

        def scale_scatter(j, b):
            jbase = jnp.zeros((16,), jnp.int32) + j * cw

            def row_body(r, carry):
                nv = plsc.load_gather(normv, [jbase + r])
                for t in range(dh // 16):
                    sl = (b, r, pl.ds(t * 16, 16))
                    rowbuf[sl] = rowbuf[sl] * nv
                return carry

            lax.fori_loop(0, cw, row_body, 0)
            pltpu.sync_copy(rowbuf.at[b], acc.at[dstv.at[j]], add=True)

        for half in range(npass):
            r0 = s * kpt + half * kh
            pltpu.sync_copy(src_hbm.at[pl.ds(c * nrows + r0, kh)], srcv)
            pltpu.sync_copy(dst_hbm.at[pl.ds(r0, kh)], dstv)
            pltpu.sync_copy(nrm_hbm.at[pl.ds(r0 * cw, kh * cw)], normv)
            issue(0, 0)

            def chunk_body(g, carry):
                issue(2 * g + 1, 1)
                wait(2 * g, 0)
                scale_scatter(2 * g, 0)

                @pl.when(g < ng - 1)
                def _():
                    issue(2 * g + 2, 0)

                wait(2 * g + 1, 1)
                scale_scatter(2 * g + 1, 1)
                return carry

            lax.fori_loop(0, ng, chunk_body, 0)

        plsc.subcore_barrier()
        pltpu.sync_copy(acc.at[pl.ds(s * pt, pt)], agg_hbm.at[pl.ds(nbase, pt)])

    mesh = plsc.VectorSubcoreMesh(core_axis_name="c", subcore_axis_name="s",
                                  num_cores=NC, num_subcores=NS)
    return pl.kernel(
        body,
        out_type=jax.ShapeDtypeStruct((NC * n_nodes, dh), jnp.float32),
        mesh=mesh,
        compiler_params=pltpu.CompilerParams(needs_layout_passes=False),
        scratch_types=[
            pltpu.VMEM_SHARED((n_nodes, dh), jnp.float32),
            pltpu.VMEM((kpt // npass, cw), jnp.int32),
            pltpu.VMEM((kpt // npass, cw), jnp.int32),
            pltpu.VMEM((kpt // npass * cw,), jnp.float32),
            pltpu.VMEM((2, cw, dh), jnp.float32),
            pltpu.SemaphoreType.DMA,
            pltpu.SemaphoreType.DMA,
        ],
    )


def _mm_first(x, w, sw, *, bm, dh):
    n, d = x.shape

    def body(x_ref, w_ref, sw_ref, g_ref, gi_ref):
        g = jnp.dot(x_ref[...], w_ref[...], preferred_element_type=jnp.float32)
        gi = g * sw_ref[...]
        g_ref[0] = g[:, :dh]
        g_ref[1] = g[:, dh:]
        gi_ref[0] = gi[:, :dh]
        gi_ref[1] = gi[:, dh:]

    return pl.pallas_call(
        body,
        grid=(n // bm,),
        in_specs=[
            pl.BlockSpec((bm, d), lambda i: (i, 0)),
            pl.BlockSpec((d, d), lambda i: (0, 0)),
            pl.BlockSpec((bm, 1), lambda i: (i, 0)),
        ],
        out_specs=[
            pl.BlockSpec((NC, bm, dh), lambda i: (0, i, 0)),
            pl.BlockSpec((NC, bm, dh), lambda i: (0, i, 0)),
        ],
        out_shape=[
            jax.ShapeDtypeStruct((NC, n, dh), jnp.float32),
            jax.ShapeDtypeStruct((NC, n, dh), jnp.float32),
        ],
    )(x, w, sw)


def _mm_mid(agg, b_prev, w, sw, *, bm, dh):
    nc, n, _ = agg.shape
    d = w.shape[0]

    def body(a_ref, b_ref, w_ref, sw_ref, g_ref, gi_ref):
        h = jnp.concatenate([a_ref[0], a_ref[1]], axis=1) + b_ref[...]
        h = jnp.maximum(h, 0.0)
        g = jnp.dot(h, w_ref[...], preferred_element_type=jnp.float32)
        gi = g * sw_ref[...]
        g_ref[0] = g[:, :dh]
        g_ref[1] = g[:, dh:]
        gi_ref[0] = gi[:, :dh]
        gi_ref[1] = gi[:, dh:]

    return pl.pallas_call(
        body,
        grid=(n // bm,),
        in_specs=[
            pl.BlockSpec((NC, bm, dh), lambda i: (0, i, 0)),
            pl.BlockSpec((1, d), lambda i: (0, 0)),
            pl.BlockSpec((d, d), lambda i: (0, 0)),
            pl.BlockSpec((bm, 1), lambda i: (i, 0)),
        ],
        out_specs=[
            pl.BlockSpec((NC, bm, dh), lambda i: (0, i, 0)),
            pl.BlockSpec((NC, bm, dh), lambda i: (0, i, 0)),
        ],
        out_shape=[
            jax.ShapeDtypeStruct((NC, n, dh), jnp.float32),
            jax.ShapeDtypeStruct((NC, n, dh), jnp.float32),
        ],
    )(agg, b_prev, w, sw)


def _mm_last(agg, b_prev, w, b_out, *, bm, dh):
    nc, n, _ = agg.shape
    d = w.shape[0]

    def body(a_ref, b_ref, w_ref, bo_ref, o_ref):
        h = jnp.concatenate([a_ref[0], a_ref[1]], axis=1) + b_ref[...]
        h = jnp.maximum(h, 0.0)
        o_ref[...] = (jnp.dot(h, w_ref[...], preferred_element_type=jnp.float32)
                      + bo_ref[...])

    return pl.pallas_call(
        body,
        grid=(n // bm,),
        in_specs=[
            pl.BlockSpec((NC, bm, dh), lambda i: (0, i, 0)),
            pl.BlockSpec((1, d), lambda i: (0, 0)),
            pl.BlockSpec((d, d), lambda i: (0, 0)),
            pl.BlockSpec((1, d), lambda i: (0, 0)),
        ],
        out_specs=pl.BlockSpec((bm, d), lambda i: (i, 0)),
        out_shape=jax.ShapeDtypeStruct((n, d), jnp.float32),
    )(agg, b_prev, w, b_out)


def kernel(x, edge_index, edge_attr, W1, b1, W2, b2, W3, b3, Wout, bout):
    n, d = x.shape
    e = edge_index.shape[1]
    dh = d // 2
    cw = 64
    kch = -(-e // (NW * CH))
    ep = NW * CH * kch
    nre = ep // CH
    nr2 = ep // cw
    npad = -(-n // 512) * 512
    bm = 512

    pad = ep - e
    src = jnp.concatenate([edge_index[0], jnp.zeros((pad,), jnp.int32)])
    dst = jnp.concatenate([edge_index[1], jnp.zeros((pad,), jnp.int32)])
    wp = jnp.concatenate([edge_attr, jnp.zeros((pad,), edge_attr.dtype)])
    src2 = jnp.stack([src, src + npad]).reshape(NC * nr2, cw)
    srcn = src.reshape(nre, CH)
    dstn = dst.reshape(nre, CH)
    dst2 = dst.reshape(nr2, cw)
    wr = wp.reshape(nre, CH)
    xp = jnp.pad(x, ((0, npad - n), (0, 0)))

    layer = _make_layer(npad, nr2, cw, dh)

    ones_t = jnp.ones((NC * npad, dh), jnp.float32)
    zeros_g = jnp.zeros((NC * npad, dh), jnp.float32)
    deg2 = layer(ones_t, zeros_g, src2, dst2, wp)
    dis, selfw = _dis_tc(deg2, npad)
    norm_r = _make_norm(npad, nre, kch)(srcn, dstn, wr, dis.reshape(npad))
    sw2 = selfw.reshape(npad, 1)

    g, gi = _mm_first(xp, W1, sw2, bm=bm, dh=dh)
    agg = layer(g.reshape(NC * npad, dh), gi.reshape(NC * npad, dh),
                src2, dst2, norm_r).reshape(NC, npad, dh)
    g, gi = _mm_mid(agg, b1.reshape(1, d), W2, sw2, bm=bm, dh=dh)
    agg = layer(g.reshape(NC * npad, dh), gi.reshape(NC * npad, dh),
                src2, dst2, norm_r).reshape(NC, npad, dh)
    g, gi = _mm_mid(agg, b2.reshape(1, d), W3, sw2, bm=bm, dh=dh)
    agg = layer(g.reshape(NC * npad, dh), gi.reshape(NC * npad, dh),
                src2, dst2, norm_r).reshape(NC, npad, dh)
    out = _mm_last(agg, b3.reshape(1, d), Wout, bout.reshape(1, d),
                   bm=bm, dh=dh)
    return out[:n]


# speedup vs baseline: 10.8460x; 1.2963x over previous
import jax
import jax.numpy as jnp
from jax import lax
from jax.experimental import pallas as pl
from jax.experimental.pallas import tpu as pltpu
from jax.experimental.pallas import tpu_sc as plsc

NC = 2
NS = 16
NW = NC * NS
CH = 128


def _make_deg(np_pad, nre, kch):
    npt = np_pad // NS

    def body(dstr_hbm, wr_hbm, deg_hbm, shared, accv, dstv, wv, redv, tmpv):
        c = lax.axis_index("c")
        s = lax.axis_index("s")
        wid = c * NS + s
        r0 = wid * kch
        pltpu.sync_copy(dstr_hbm.at[pl.ds(r0, kch)], dstv)
        pltpu.sync_copy(wr_hbm.at[pl.ds(r0, kch)], wv)

        def zb(i, carry):
            accv[pl.ds(i * 16, 16)] = jnp.zeros((16,), jnp.float32)
            return carry

        lax.fori_loop(0, np_pad // 16, zb, 0)
        for j in range(kch):
            for k in range(CH // 16):
                sl = pl.ds(k * 16, 16)
                plsc.addupdate_scatter(accv, [dstv[j, sl]], wv[j, sl])
        pltpu.sync_copy(accv, shared.at[s])
        plsc.subcore_barrier()

        base = s * npt
        pltpu.sync_copy(shared.at[0, pl.ds(base, npt)], redv)
        for t in range(1, NS):
            pltpu.sync_copy(shared.at[t, pl.ds(base, npt)], tmpv)

            def ab(i, carry):
                sl = pl.ds(i * 16, 16)
                redv[sl] = redv[sl] + tmpv[sl]
                return carry

            lax.fori_loop(0, npt // 16, ab, 0)
        pltpu.sync_copy(redv, deg_hbm.at[pl.ds(c * np_pad + base, npt)])

    mesh = plsc.VectorSubcoreMesh(core_axis_name="c", subcore_axis_name="s",
                                  num_cores=NC, num_subcores=NS)
    return pl.kernel(
        body,
        out_type=jax.ShapeDtypeStruct((NC * np_pad,), jnp.float32),
        mesh=mesh,
        compiler_params=pltpu.CompilerParams(needs_layout_passes=False),
        scratch_types=[
            pltpu.VMEM_SHARED((NS, np_pad), jnp.float32),
            pltpu.VMEM((np_pad,), jnp.float32),
            pltpu.VMEM((kch, CH), jnp.int32),
            pltpu.VMEM((kch, CH), jnp.float32),
            pltpu.VMEM((npt,), jnp.float32),
            pltpu.VMEM((npt,), jnp.float32),
        ],
    )


def _dis_tc(deg2, np_pad):
    def body(d_ref, dis_ref, sw_ref):
        d = d_ref[...]
        dsum = d[0] + d[1] + 1.0
        dis = lax.rsqrt(jnp.maximum(dsum, 1e-12))
        dis_ref[...] = dis.reshape(1, np_pad)
        sw_ref[...] = (dis * dis).reshape(1, np_pad)

    return pl.pallas_call(
        body,
        out_shape=[
            jax.ShapeDtypeStruct((1, np_pad), jnp.float32),
            jax.ShapeDtypeStruct((1, np_pad), jnp.float32),
        ],
    )(deg2)


def _make_norm(np_pad, nre, kch):
    def body(src_hbm, dstr_hbm, wr_hbm, dis_hbm, norm_hbm,
             disv, srcv, dstnv, wv, normv):
        c = lax.axis_index("c")
        s = lax.axis_index("s")
        wid = c * NS + s
        r0 = wid * kch

        pltpu.sync_copy(dis_hbm, disv)
        pltpu.sync_copy(src_hbm.at[pl.ds(r0, kch)], srcv)
        pltpu.sync_copy(dstr_hbm.at[pl.ds(r0, kch)], dstnv)
        pltpu.sync_copy(wr_hbm.at[pl.ds(r0, kch)], wv)
        for j in range(kch):
            for k in range(CH // 16):
                sl = pl.ds(k * 16, 16)
                nrm = (plsc.load_gather(disv, [srcv[j, sl]]) * wv[j, sl]
                       * plsc.load_gather(disv, [dstnv[j, sl]]))
                normv[pl.ds(j * CH + k * 16, 16)] = nrm
        pltpu.sync_copy(normv, norm_hbm.at[pl.ds(r0 * CH, kch * CH)])

    mesh = plsc.VectorSubcoreMesh(core_axis_name="c", subcore_axis_name="s",
                                  num_cores=NC, num_subcores=NS)
    return pl.kernel(
        body,
        out_type=jax.ShapeDtypeStruct((nre * CH,), jnp.float32),
        mesh=mesh,
        compiler_params=pltpu.CompilerParams(needs_layout_passes=False),
        scratch_types=[
            pltpu.VMEM((np_pad,), jnp.float32),
            pltpu.VMEM((kch, CH), jnp.int32),
            pltpu.VMEM((kch, CH), jnp.int32),
            pltpu.VMEM((kch, CH), jnp.float32),
            pltpu.VMEM((kch * CH,), jnp.float32),
        ],
    )


def _make_layer(n_nodes, nrows, cw, dh):
    pt = n_nodes // NS
    kpt = nrows // NS
    npass = 2
    kh = kpt // npass
    ng = kh // 2

    def body(g_hbm, gi_hbm, src_hbm, dst_hbm, nrm_hbm, agg_hbm,
             acc, srcv, dstv, normv, rowbuf, sem0, sem1):
        c = lax.axis_index("c")
        s = lax.axis_index("s")
        nbase = c * n_nodes + s * pt

        pltpu.sync_copy(gi_hbm.at[pl.ds(nbase, pt)], acc.at[pl.ds(s * pt, pt)])
        plsc.subcore_barrier()

        sems = [sem0, sem1]

        def issue(j, b):
            return pltpu.async_copy(g_hbm.at[srcv.at[j]], rowbuf.at[b],
                                    sems[b])

        def wait(j, b):
            pltpu.make_async_copy(g_hbm.at[srcv.at[j]], rowbuf.at[b],
                                  sems[b]).wait()

        def scale_scatter(j, b):
            jbase = jnp.zeros((16,), jnp.int32) + j * cw

            def row_body(r, carry):
                nv = plsc.load_gather(normv, [jbase + r])
                for t in range(dh // 16):
                    sl = (b, r, pl.ds(t * 16, 16))
                    rowbuf[sl] = rowbuf[sl] * nv
                return carry

            lax.fori_loop(0, cw, row_body, 0)
            pltpu.sync_copy(rowbuf.at[b], acc.at[dstv.at[j]], add=True)

        for half in range(npass):
            r0 = s * kpt + half * kh
            pltpu.sync_copy(src_hbm.at[pl.ds(c * nrows + r0, kh)], srcv)
            pltpu.sync_copy(dst_hbm.at[pl.ds(r0, kh)], dstv)
            pltpu.sync_copy(nrm_hbm.at[pl.ds(r0 * cw, kh * cw)], normv)
            issue(0, 0)

            def chunk_body(g, carry):
                issue(2 * g + 1, 1)
                wait(2 * g, 0)
                scale_scatter(2 * g, 0)

                @pl.when(g < ng - 1)
                def _():
                    issue(2 * g + 2, 0)

                wait(2 * g + 1, 1)
                scale_scatter(2 * g + 1, 1)
                return carry

            lax.fori_loop(0, ng, chunk_body, 0)

        plsc.subcore_barrier()
        pltpu.sync_copy(acc.at[pl.ds(s * pt, pt)], agg_hbm.at[pl.ds(nbase, pt)])

    mesh = plsc.VectorSubcoreMesh(core_axis_name="c", subcore_axis_name="s",
                                  num_cores=NC, num_subcores=NS)
    return pl.kernel(
        body,
        out_type=jax.ShapeDtypeStruct((NC * n_nodes, dh), jnp.float32),
        mesh=mesh,
        compiler_params=pltpu.CompilerParams(needs_layout_passes=False),
        scratch_types=[
            pltpu.VMEM_SHARED((n_nodes, dh), jnp.float32),
            pltpu.VMEM((kpt // npass, cw), jnp.int32),
            pltpu.VMEM((kpt // npass, cw), jnp.int32),
            pltpu.VMEM((kpt // npass * cw,), jnp.float32),
            pltpu.VMEM((2, cw, dh), jnp.float32),
            pltpu.SemaphoreType.DMA,
            pltpu.SemaphoreType.DMA,
        ],
    )


def _mm_first(x, w, sw, *, bm, dh):
    n, d = x.shape

    def body(x_ref, w_ref, sw_ref, g_ref, gi_ref):
        g = jnp.dot(x_ref[...], w_ref[...], preferred_element_type=jnp.float32)
        gi = g * sw_ref[...]
        g_ref[0] = g[:, :dh]
        g_ref[1] = g[:, dh:]
        gi_ref[0] = gi[:, :dh]
        gi_ref[1] = gi[:, dh:]

    return pl.pallas_call(
        body,
        grid=(n // bm,),
        in_specs=[
            pl.BlockSpec((bm, d), lambda i: (i, 0)),
            pl.BlockSpec((d, d), lambda i: (0, 0)),
            pl.BlockSpec((bm, 1), lambda i: (i, 0)),
        ],
        out_specs=[
            pl.BlockSpec((NC, bm, dh), lambda i: (0, i, 0)),
            pl.BlockSpec((NC, bm, dh), lambda i: (0, i, 0)),
        ],
        out_shape=[
            jax.ShapeDtypeStruct((NC, n, dh), jnp.float32),
            jax.ShapeDtypeStruct((NC, n, dh), jnp.float32),
        ],
    )(x, w, sw)


def _mm_mid(agg, b_prev, w, sw, *, bm, dh):
    nc, n, _ = agg.shape
    d = w.shape[0]

    def body(a_ref, b_ref, w_ref, sw_ref, g_ref, gi_ref):
        h = jnp.concatenate([a_ref[0], a_ref[1]], axis=1) + b_ref[...]
        h = jnp.maximum(h, 0.0)
        g = jnp.dot(h, w_ref[...], preferred_element_type=jnp.float32)
        gi = g * sw_ref[...]
        g_ref[0] = g[:, :dh]
        g_ref[1] = g[:, dh:]
        gi_ref[0] = gi[:, :dh]
        gi_ref[1] = gi[:, dh:]

    return pl.pallas_call(
        body,
        grid=(n // bm,),
        in_specs=[
            pl.BlockSpec((NC, bm, dh), lambda i: (0, i, 0)),
            pl.BlockSpec((1, d), lambda i: (0, 0)),
            pl.BlockSpec((d, d), lambda i: (0, 0)),
            pl.BlockSpec((bm, 1), lambda i: (i, 0)),
        ],
        out_specs=[
            pl.BlockSpec((NC, bm, dh), lambda i: (0, i, 0)),
            pl.BlockSpec((NC, bm, dh), lambda i: (0, i, 0)),
        ],
        out_shape=[
            jax.ShapeDtypeStruct((NC, n, dh), jnp.float32),
            jax.ShapeDtypeStruct((NC, n, dh), jnp.float32),
        ],
    )(agg, b_prev, w, sw)


def _mm_last(agg, b_prev, w, b_out, *, bm, dh):
    nc, n, _ = agg.shape
    d = w.shape[0]

    def body(a_ref, b_ref, w_ref, bo_ref, o_ref):
        h = jnp.concatenate([a_ref[0], a_ref[1]], axis=1) + b_ref[...]
        h = jnp.maximum(h, 0.0)
        o_ref[...] = (jnp.dot(h, w_ref[...], preferred_element_type=jnp.float32)
                      + bo_ref[...])

    return pl.pallas_call(
        body,
        grid=(n // bm,),
        in_specs=[
            pl.BlockSpec((NC, bm, dh), lambda i: (0, i, 0)),
            pl.BlockSpec((1, d), lambda i: (0, 0)),
            pl.BlockSpec((d, d), lambda i: (0, 0)),
            pl.BlockSpec((1, d), lambda i: (0, 0)),
        ],
        out_specs=pl.BlockSpec((bm, d), lambda i: (i, 0)),
        out_shape=jax.ShapeDtypeStruct((n, d), jnp.float32),
    )(agg, b_prev, w, b_out)


def kernel(x, edge_index, edge_attr, W1, b1, W2, b2, W3, b3, Wout, bout):
    n, d = x.shape
    e = edge_index.shape[1]
    dh = d // 2
    cw = 64
    kch = -(-e // (NW * CH))
    ep = NW * CH * kch
    nre = ep // CH
    nr2 = ep // cw
    npad = -(-n // 512) * 512
    bm = 512

    pad = ep - e
    src = jnp.concatenate([edge_index[0], jnp.zeros((pad,), jnp.int32)])
    dst = jnp.concatenate([edge_index[1], jnp.zeros((pad,), jnp.int32)])
    wp = jnp.concatenate([edge_attr, jnp.zeros((pad,), edge_attr.dtype)])
    src2 = jnp.stack([src, src + npad]).reshape(NC * nr2, cw)
    srcn = src.reshape(nre, CH)
    dstn = dst.reshape(nre, CH)
    dst2 = dst.reshape(nr2, cw)
    wr = wp.reshape(nre, CH)
    xp = jnp.pad(x, ((0, npad - n), (0, 0)))

    layer = _make_layer(npad, nr2, cw, dh)

    degp = _make_deg(npad, nre, kch)(dstn, wr)
    dis, selfw = _dis_tc(degp.reshape(NC, npad), npad)
    norm_r = _make_norm(npad, nre, kch)(srcn, dstn, wr, dis.reshape(npad))
    sw2 = selfw.reshape(npad, 1)

    g, gi = _mm_first(xp, W1, sw2, bm=bm, dh=dh)
    agg = layer(g.reshape(NC * npad, dh), gi.reshape(NC * npad, dh),
                src2, dst2, norm_r).reshape(NC, npad, dh)
    g, gi = _mm_mid(agg, b1.reshape(1, d), W2, sw2, bm=bm, dh=dh)
    agg = layer(g.reshape(NC * npad, dh), gi.reshape(NC * npad, dh),
                src2, dst2, norm_r).reshape(NC, npad, dh)
    g, gi = _mm_mid(agg, b2.reshape(1, d), W3, sw2, bm=bm, dh=dh)
    agg = layer(g.reshape(NC * npad, dh), gi.reshape(NC * npad, dh),
                src2, dst2, norm_r).reshape(NC, npad, dh)
    out = _mm_last(agg, b3.reshape(1, d), Wout, bout.reshape(1, d),
                   bm=bm, dh=dh)
    return out[:n]


# speedup vs baseline: 11.2014x; 1.0328x over previous
import jax
import jax.numpy as jnp
from jax import lax
from jax.experimental import pallas as pl
from jax.experimental.pallas import tpu as pltpu
from jax.experimental.pallas import tpu_sc as plsc

NC = 2
NS = 16
NW = NC * NS
CH = 128


def _make_deg(np_pad, nre, kch):
    npt = np_pad // NS

    def body(dstr_hbm, wr_hbm, deg_hbm, shared, accv, dstv, wv, redv, tmpv):
        c = lax.axis_index("c")
        s = lax.axis_index("s")
        wid = c * NS + s
        r0 = wid * kch
        pltpu.sync_copy(dstr_hbm.at[pl.ds(r0, kch)], dstv)
        pltpu.sync_copy(wr_hbm.at[pl.ds(r0, kch)], wv)

        def zb(i, carry):
            accv[pl.ds(i * 16, 16)] = jnp.zeros((16,), jnp.float32)
            return carry

        lax.fori_loop(0, np_pad // 16, zb, 0)
        for j in range(kch):
            for k in range(CH // 16):
                sl = pl.ds(k * 16, 16)
                plsc.addupdate_scatter(accv, [dstv[j, sl]], wv[j, sl])
        pltpu.sync_copy(accv, shared.at[s])
        plsc.subcore_barrier()

        base = s * npt
        pltpu.sync_copy(shared.at[0, pl.ds(base, npt)], redv)
        for t in range(1, NS):
            pltpu.sync_copy(shared.at[t, pl.ds(base, npt)], tmpv)

            def ab(i, carry):
                sl = pl.ds(i * 16, 16)
                redv[sl] = redv[sl] + tmpv[sl]
                return carry

            lax.fori_loop(0, npt // 16, ab, 0)
        pltpu.sync_copy(redv, deg_hbm.at[pl.ds(c * np_pad + base, npt)])

    mesh = plsc.VectorSubcoreMesh(core_axis_name="c", subcore_axis_name="s",
                                  num_cores=NC, num_subcores=NS)
    return pl.kernel(
        body,
        out_type=jax.ShapeDtypeStruct((NC * np_pad,), jnp.float32),
        mesh=mesh,
        compiler_params=pltpu.CompilerParams(needs_layout_passes=False),
        scratch_types=[
            pltpu.VMEM_SHARED((NS, np_pad), jnp.float32),
            pltpu.VMEM((np_pad,), jnp.float32),
            pltpu.VMEM((kch, CH), jnp.int32),
            pltpu.VMEM((kch, CH), jnp.float32),
            pltpu.VMEM((npt,), jnp.float32),
            pltpu.VMEM((npt,), jnp.float32),
        ],
    )


def _dis_tc(deg2, np_pad):
    def body(d_ref, dis_ref, sw_ref):
        d = d_ref[...]
        dsum = d[0] + d[1] + 1.0
        dis = lax.rsqrt(jnp.maximum(dsum, 1e-12))
        dis_ref[...] = dis.reshape(1, np_pad)
        sw_ref[...] = (dis * dis).reshape(1, np_pad)

    return pl.pallas_call(
        body,
        out_shape=[
            jax.ShapeDtypeStruct((1, np_pad), jnp.float32),
            jax.ShapeDtypeStruct((1, np_pad), jnp.float32),
        ],
    )(deg2)


def _make_norm(np_pad, nre, kch):
    def body(src_hbm, dstr_hbm, wr_hbm, dis_hbm, norm_hbm,
             disv, srcv, dstnv, wv, normv):
        c = lax.axis_index("c")
        s = lax.axis_index("s")
        wid = c * NS + s
        r0 = wid * kch

        pltpu.sync_copy(dis_hbm, disv)
        pltpu.sync_copy(src_hbm.at[pl.ds(r0, kch)], srcv)
        pltpu.sync_copy(dstr_hbm.at[pl.ds(r0, kch)], dstnv)
        pltpu.sync_copy(wr_hbm.at[pl.ds(r0, kch)], wv)
        for j in range(kch):
            for k in range(CH // 16):
                sl = pl.ds(k * 16, 16)
                nrm = (plsc.load_gather(disv, [srcv[j, sl]]) * wv[j, sl]
                       * plsc.load_gather(disv, [dstnv[j, sl]]))
                normv[pl.ds(j * CH + k * 16, 16)] = nrm
        pltpu.sync_copy(normv, norm_hbm.at[pl.ds(r0 * CH, kch * CH)])

    mesh = plsc.VectorSubcoreMesh(core_axis_name="c", subcore_axis_name="s",
                                  num_cores=NC, num_subcores=NS)
    return pl.kernel(
        body,
        out_type=jax.ShapeDtypeStruct((nre * CH,), jnp.float32),
        mesh=mesh,
        compiler_params=pltpu.CompilerParams(needs_layout_passes=False),
        scratch_types=[
            pltpu.VMEM((np_pad,), jnp.float32),
            pltpu.VMEM((kch, CH), jnp.int32),
            pltpu.VMEM((kch, CH), jnp.int32),
            pltpu.VMEM((kch, CH), jnp.float32),
            pltpu.VMEM((kch * CH,), jnp.float32),
        ],
    )


def _make_layer(n_nodes, nrows, cw, dh):
    pt = n_nodes // NS
    kpt = nrows // NS
    npass = 4
    kh = kpt // npass
    ngrp = kh // 3
    tail = kh - 3 * ngrp

    def body(g_hbm, gi_hbm, src_hbm, dst_hbm, nrm_hbm, agg_hbm,
             acc, srcv, dstv, normv, rowbuf,
             g0, g1, g2, s0, s1, s2):
        c = lax.axis_index("c")
        s = lax.axis_index("s")
        nbase = c * n_nodes + s * pt

        pltpu.sync_copy(gi_hbm.at[pl.ds(nbase, pt)], acc.at[pl.ds(s * pt, pt)])
        plsc.subcore_barrier()

        gsem = [g0, g1, g2]
        ssem = [s0, s1, s2]

        def issue(j, b):
            pltpu.async_copy(g_hbm.at[srcv.at[j]], rowbuf.at[b], gsem[b])

        def wait_gather(j, b):
            pltpu.make_async_copy(g_hbm.at[srcv.at[j]], rowbuf.at[b],
                                  gsem[b]).wait()

        def issue_scatter(j, b):
            pltpu.async_copy(rowbuf.at[b], acc.at[dstv.at[j]], ssem[b],
                             add=True)

        def wait_scatter(j, b):
            pltpu.make_async_copy(rowbuf.at[b], acc.at[dstv.at[j]],
                                  ssem[b]).wait()

        def scale(j, b):
            jbase = jnp.zeros((16,), jnp.int32) + j * cw

            def row_body(r2, carry):
                for dr in range(2):
                    r = 2 * r2 + dr
                    nv = plsc.load_gather(normv, [jbase + r])
                    for t in range(dh // 16):
                        sl = (b, r, pl.ds(t * 16, 16))
                        rowbuf[sl] = rowbuf[sl] * nv
                return carry

            lax.fori_loop(0, cw // 2, row_body, 0)

        def step(j, b, first_group):
            wait_gather(j, b)
            scale(j, b)
            bp = (b + 2) % 3
            if first_group is None or not first_group:
                wait_scatter(j, bp)
            else:

                @pl.when(j >= 1)
                def _():
                    wait_scatter(j, bp)

            @pl.when(j + 2 < kh)
            def _():
                issue(j + 2, bp)

            issue_scatter(j, b)

        for half in range(npass):
            r0 = s * kpt + half * kh
            pltpu.sync_copy(src_hbm.at[pl.ds(c * nrows + r0, kh)], srcv)
            pltpu.sync_copy(dst_hbm.at[pl.ds(r0, kh)], dstv)
            pltpu.sync_copy(nrm_hbm.at[pl.ds(r0 * cw, kh * cw)], normv)
            issue(0, 0)
            issue(1, 1)

            def grp(gi, carry):
                for b in range(3):
                    step(3 * gi + b, b, first_group=(b == 0))
                return carry

            lax.fori_loop(0, ngrp, grp, 0)
            for b in range(tail):
                step(3 * ngrp + b, b, first_group=False)
            wait_scatter(kh - 1, (kh - 1) % 3)

        plsc.subcore_barrier()
        pltpu.sync_copy(acc.at[pl.ds(s * pt, pt)], agg_hbm.at[pl.ds(nbase, pt)])

    mesh = plsc.VectorSubcoreMesh(core_axis_name="c", subcore_axis_name="s",
                                  num_cores=NC, num_subcores=NS)
    return pl.kernel(
        body,
        out_type=jax.ShapeDtypeStruct((NC * n_nodes, dh), jnp.float32),
        mesh=mesh,
        compiler_params=pltpu.CompilerParams(needs_layout_passes=False),
        scratch_types=[
            pltpu.VMEM_SHARED((n_nodes, dh), jnp.float32),
            pltpu.VMEM((kpt // npass, cw), jnp.int32),
            pltpu.VMEM((kpt // npass, cw), jnp.int32),
            pltpu.VMEM((kpt // npass * cw,), jnp.float32),
            pltpu.VMEM((3, cw, dh), jnp.float32),
            pltpu.SemaphoreType.DMA,
            pltpu.SemaphoreType.DMA,
            pltpu.SemaphoreType.DMA,
            pltpu.SemaphoreType.DMA,
            pltpu.SemaphoreType.DMA,
            pltpu.SemaphoreType.DMA,
        ],
    )


def _mm_first(x, w, sw, *, bm, dh):
    n, d = x.shape

    def body(x_ref, w_ref, sw_ref, g_ref, gi_ref):
        g = jnp.dot(x_ref[...], w_ref[...], preferred_element_type=jnp.float32)
        gi = g * sw_ref[...]
        g_ref[0] = g[:, :dh]
        g_ref[1] = g[:, dh:]
        gi_ref[0] = gi[:, :dh]
        gi_ref[1] = gi[:, dh:]

    return pl.pallas_call(
        body,
        grid=(n // bm,),
        in_specs=[
            pl.BlockSpec((bm, d), lambda i: (i, 0)),
            pl.BlockSpec((d, d), lambda i: (0, 0)),
            pl.BlockSpec((bm, 1), lambda i: (i, 0)),
        ],
        out_specs=[
            pl.BlockSpec((NC, bm, dh), lambda i: (0, i, 0)),
            pl.BlockSpec((NC, bm, dh), lambda i: (0, i, 0)),
        ],
        out_shape=[
            jax.ShapeDtypeStruct((NC, n, dh), jnp.float32),
            jax.ShapeDtypeStruct((NC, n, dh), jnp.float32),
        ],
    )(x, w, sw)


def _mm_mid(agg, b_prev, w, sw, *, bm, dh):
    nc, n, _ = agg.shape
    d = w.shape[0]

    def body(a_ref, b_ref, w_ref, sw_ref, g_ref, gi_ref):
        h = jnp.concatenate([a_ref[0], a_ref[1]], axis=1) + b_ref[...]
        h = jnp.maximum(h, 0.0)
        g = jnp.dot(h, w_ref[...], preferred_element_type=jnp.float32)
        gi = g * sw_ref[...]
        g_ref[0] = g[:, :dh]
        g_ref[1] = g[:, dh:]
        gi_ref[0] = gi[:, :dh]
        gi_ref[1] = gi[:, dh:]

    return pl.pallas_call(
        body,
        grid=(n // bm,),
        in_specs=[
            pl.BlockSpec((NC, bm, dh), lambda i: (0, i, 0)),
            pl.BlockSpec((1, d), lambda i: (0, 0)),
            pl.BlockSpec((d, d), lambda i: (0, 0)),
            pl.BlockSpec((bm, 1), lambda i: (i, 0)),
        ],
        out_specs=[
            pl.BlockSpec((NC, bm, dh), lambda i: (0, i, 0)),
            pl.BlockSpec((NC, bm, dh), lambda i: (0, i, 0)),
        ],
        out_shape=[
            jax.ShapeDtypeStruct((NC, n, dh), jnp.float32),
            jax.ShapeDtypeStruct((NC, n, dh), jnp.float32),
        ],
    )(agg, b_prev, w, sw)


def _mm_last(agg, b_prev, w, b_out, *, bm, dh):
    nc, n, _ = agg.shape
    d = w.shape[0]

    def body(a_ref, b_ref, w_ref, bo_ref, o_ref):
        h = jnp.concatenate([a_ref[0], a_ref[1]], axis=1) + b_ref[...]
        h = jnp.maximum(h, 0.0)
        o_ref[...] = (jnp.dot(h, w_ref[...], preferred_element_type=jnp.float32)
                      + bo_ref[...])

    return pl.pallas_call(
        body,
        grid=(n // bm,),
        in_specs=[
            pl.BlockSpec((NC, bm, dh), lambda i: (0, i, 0)),
            pl.BlockSpec((1, d), lambda i: (0, 0)),
            pl.BlockSpec((d, d), lambda i: (0, 0)),
            pl.BlockSpec((1, d), lambda i: (0, 0)),
        ],
        out_specs=pl.BlockSpec((bm, d), lambda i: (i, 0)),
        out_shape=jax.ShapeDtypeStruct((n, d), jnp.float32),
    )(agg, b_prev, w, b_out)


def kernel(x, edge_index, edge_attr, W1, b1, W2, b2, W3, b3, Wout, bout):
    n, d = x.shape
    e = edge_index.shape[1]
    dh = d // 2
    cw = 64
    kch = -(-e // (NW * CH))
    ep = NW * CH * kch
    nre = ep // CH
    nr2 = ep // cw
    npad = -(-n // 512) * 512
    bm = 512

    pad = ep - e
    src = jnp.concatenate([edge_index[0], jnp.zeros((pad,), jnp.int32)])
    dst = jnp.concatenate([edge_index[1], jnp.zeros((pad,), jnp.int32)])
    wp = jnp.concatenate([edge_attr, jnp.zeros((pad,), edge_attr.dtype)])
    src2 = jnp.stack([src, src + npad]).reshape(NC * nr2, cw)
    srcn = src.reshape(nre, CH)
    dstn = dst.reshape(nre, CH)
    dst2 = dst.reshape(nr2, cw)
    wr = wp.reshape(nre, CH)
    xp = jnp.pad(x, ((0, npad - n), (0, 0)))

    layer = _make_layer(npad, nr2, cw, dh)

    degp = _make_deg(npad, nre, kch)(dstn, wr)
    dis, selfw = _dis_tc(degp.reshape(NC, npad), npad)
    norm_r = _make_norm(npad, nre, kch)(srcn, dstn, wr, dis.reshape(npad))
    sw2 = selfw.reshape(npad, 1)

    g, gi = _mm_first(xp, W1, sw2, bm=bm, dh=dh)
    agg = layer(g.reshape(NC * npad, dh), gi.reshape(NC * npad, dh),
                src2, dst2, norm_r).reshape(NC, npad, dh)
    g, gi = _mm_mid(agg, b1.reshape(1, d), W2, sw2, bm=bm, dh=dh)
    agg = layer(g.reshape(NC * npad, dh), gi.reshape(NC * npad, dh),
                src2, dst2, norm_r).reshape(NC, npad, dh)
    g, gi = _mm_mid(agg, b2.reshape(1, d), W3, sw2, bm=bm, dh=dh)
    agg = layer(g.reshape(NC * npad, dh), gi.reshape(NC * npad, dh),
                src2, dst2, norm_r).reshape(NC, npad, dh)
    out = _mm_last(agg, b3.reshape(1, d), Wout, bout.reshape(1, d),
                   bm=bm, dh=dh)
    return out[:n]


# speedup vs baseline: 11.4937x; 1.0261x over previous
import jax
import jax.numpy as jnp
from jax import lax
from jax.experimental import pallas as pl
from jax.experimental.pallas import tpu as pltpu
from jax.experimental.pallas import tpu_sc as plsc

NC = 2
NS = 16
NW = NC * NS
CH = 128


def _make_deg(np_pad, nre, kch):
    npt = np_pad // NS

    def body(dstr_hbm, wr_hbm, deg_hbm, shared, accv, dstv, wv, redv, tmpv):
        c = lax.axis_index("c")
        s = lax.axis_index("s")
        wid = c * NS + s
        r0 = wid * kch
        pltpu.sync_copy(dstr_hbm.at[pl.ds(r0, kch)], dstv)
        pltpu.sync_copy(wr_hbm.at[pl.ds(r0, kch)], wv)

        def zb(i, carry):
            accv[pl.ds(i * 16, 16)] = jnp.zeros((16,), jnp.float32)
            return carry

        lax.fori_loop(0, np_pad // 16, zb, 0)
        for j in range(kch):
            for k in range(CH // 16):
                sl = pl.ds(k * 16, 16)
                plsc.addupdate_scatter(accv, [dstv[j, sl]], wv[j, sl])
        pltpu.sync_copy(accv, shared.at[s])
        plsc.subcore_barrier()

        base = s * npt
        pltpu.sync_copy(shared.at[0, pl.ds(base, npt)], redv)
        for t in range(1, NS):
            pltpu.sync_copy(shared.at[t, pl.ds(base, npt)], tmpv)

            def ab(i, carry):
                sl = pl.ds(i * 16, 16)
                redv[sl] = redv[sl] + tmpv[sl]
                return carry

            lax.fori_loop(0, npt // 16, ab, 0)
        pltpu.sync_copy(redv, deg_hbm.at[pl.ds(c * np_pad + base, npt)])

    mesh = plsc.VectorSubcoreMesh(core_axis_name="c", subcore_axis_name="s",
                                  num_cores=NC, num_subcores=NS)
    return pl.kernel(
        body,
        out_type=jax.ShapeDtypeStruct((NC * np_pad,), jnp.float32),
        mesh=mesh,
        compiler_params=pltpu.CompilerParams(needs_layout_passes=False),
        scratch_types=[
            pltpu.VMEM_SHARED((NS, np_pad), jnp.float32),
            pltpu.VMEM((np_pad,), jnp.float32),
            pltpu.VMEM((kch, CH), jnp.int32),
            pltpu.VMEM((kch, CH), jnp.float32),
            pltpu.VMEM((npt,), jnp.float32),
            pltpu.VMEM((npt,), jnp.float32),
        ],
    )


def _dis_tc(deg2, np_pad):
    def body(d_ref, dis_ref, sw_ref):
        d = d_ref[...]
        dsum = d[0] + d[1] + 1.0
        dis = lax.rsqrt(jnp.maximum(dsum, 1e-12))
        dis_ref[...] = dis.reshape(1, np_pad)
        sw_ref[...] = (dis * dis).reshape(1, np_pad)

    return pl.pallas_call(
        body,
        out_shape=[
            jax.ShapeDtypeStruct((1, np_pad), jnp.float32),
            jax.ShapeDtypeStruct((1, np_pad), jnp.float32),
        ],
    )(deg2)


def _make_norm(np_pad, nre, kch):
    def body(src_hbm, dstr_hbm, wr_hbm, dis_hbm, norm_hbm,
             disv, srcv, dstnv, wv, normv):
        c = lax.axis_index("c")
        s = lax.axis_index("s")
        wid = c * NS + s
        r0 = wid * kch

        pltpu.sync_copy(dis_hbm, disv)
        pltpu.sync_copy(src_hbm.at[pl.ds(r0, kch)], srcv)
        pltpu.sync_copy(dstr_hbm.at[pl.ds(r0, kch)], dstnv)
        pltpu.sync_copy(wr_hbm.at[pl.ds(r0, kch)], wv)
        for j in range(kch):
            for k in range(CH // 16):
                sl = pl.ds(k * 16, 16)
                nrm = (plsc.load_gather(disv, [srcv[j, sl]]) * wv[j, sl]
                       * plsc.load_gather(disv, [dstnv[j, sl]]))
                normv[pl.ds(j * CH + k * 16, 16)] = nrm
        pltpu.sync_copy(normv, norm_hbm.at[pl.ds(r0 * CH, kch * CH)])

    mesh = plsc.VectorSubcoreMesh(core_axis_name="c", subcore_axis_name="s",
                                  num_cores=NC, num_subcores=NS)
    return pl.kernel(
        body,
        out_type=jax.ShapeDtypeStruct((nre * CH,), jnp.float32),
        mesh=mesh,
        compiler_params=pltpu.CompilerParams(needs_layout_passes=False),
        scratch_types=[
            pltpu.VMEM((np_pad,), jnp.float32),
            pltpu.VMEM((kch, CH), jnp.int32),
            pltpu.VMEM((kch, CH), jnp.int32),
            pltpu.VMEM((kch, CH), jnp.float32),
            pltpu.VMEM((kch * CH,), jnp.float32),
        ],
    )


def _make_layer(n_nodes, nrows, cw, dh):
    pt = n_nodes // NS
    kpt = nrows // NS
    npass = 4
    kh = kpt // npass
    ngrp = kh // 3
    tail = kh - 3 * ngrp

    def body(g_hbm, gi_hbm, src_hbm, dst_hbm, nrm_hbm, agg_hbm,
             acc, srcv, dstv, normv, rowbuf,
             g0, g1, g2, s0, s1, s2):
        c = lax.axis_index("c")
        s = lax.axis_index("s")
        nbase = c * n_nodes + s * pt

        pltpu.sync_copy(gi_hbm.at[pl.ds(nbase, pt)], acc.at[pl.ds(s * pt, pt)])
        plsc.subcore_barrier()

        gsem = [g0, g1, g2]
        ssem = [s0, s1, s2]

        def issue(j, b):
            pltpu.async_copy(g_hbm.at[srcv.at[j]], rowbuf.at[b], gsem[b])

        def wait_gather(j, b):
            pltpu.make_async_copy(g_hbm.at[srcv.at[j]], rowbuf.at[b],
                                  gsem[b]).wait()

        def issue_scatter(j, b):
            pltpu.async_copy(rowbuf.at[b], acc.at[dstv.at[j]], ssem[b],
                             add=True)

        def wait_scatter(j, b):
            pltpu.make_async_copy(rowbuf.at[b], acc.at[dstv.at[j]],
                                  ssem[b]).wait()

        def scale(j, b):
            jbase = jnp.zeros((16,), jnp.int32) + j * cw

            def row_body(r4, carry):
                rr = r4 * 4
                nvs = [plsc.load_gather(normv, [jbase + (rr + dr)])
                       for dr in range(4)]
                for dr in range(4):
                    for t in range(dh // 16):
                        sl = (b, rr + dr, pl.ds(t * 16, 16))
                        rowbuf[sl] = rowbuf[sl] * nvs[dr]
                return carry

            lax.fori_loop(0, cw // 4, row_body, 0)

        def step(j, b, first_group):
            wait_gather(j, b)
            scale(j, b)
            bp = (b + 2) % 3
            if first_group is None or not first_group:
                wait_scatter(j, bp)
            else:

                @pl.when(j >= 1)
                def _():
                    wait_scatter(j, bp)

            @pl.when(j + 2 < kh)
            def _():
                issue(j + 2, bp)

            issue_scatter(j, b)

        for half in range(npass):
            r0 = s * kpt + half * kh
            pltpu.sync_copy(src_hbm.at[pl.ds(c * nrows + r0, kh)], srcv)
            pltpu.sync_copy(dst_hbm.at[pl.ds(r0, kh)], dstv)
            pltpu.sync_copy(nrm_hbm.at[pl.ds(r0 * cw, kh * cw)], normv)
            issue(0, 0)
            issue(1, 1)

            def grp(gi, carry):
                for b in range(3):
                    step(3 * gi + b, b, first_group=(b == 0))
                return carry

            lax.fori_loop(0, ngrp, grp, 0)
            for b in range(tail):
                step(3 * ngrp + b, b, first_group=False)
            wait_scatter(kh - 1, (kh - 1) % 3)

        plsc.subcore_barrier()
        pltpu.sync_copy(acc.at[pl.ds(s * pt, pt)], agg_hbm.at[pl.ds(nbase, pt)])

    mesh = plsc.VectorSubcoreMesh(core_axis_name="c", subcore_axis_name="s",
                                  num_cores=NC, num_subcores=NS)
    return pl.kernel(
        body,
        out_type=jax.ShapeDtypeStruct((NC * n_nodes, dh), jnp.float32),
        mesh=mesh,
        compiler_params=pltpu.CompilerParams(needs_layout_passes=False),
        scratch_types=[
            pltpu.VMEM_SHARED((n_nodes, dh), jnp.float32),
            pltpu.VMEM((kpt // npass, cw), jnp.int32),
            pltpu.VMEM((kpt // npass, cw), jnp.int32),
            pltpu.VMEM((kpt // npass * cw,), jnp.float32),
            pltpu.VMEM((3, cw, dh), jnp.float32),
            pltpu.SemaphoreType.DMA,
            pltpu.SemaphoreType.DMA,
            pltpu.SemaphoreType.DMA,
            pltpu.SemaphoreType.DMA,
            pltpu.SemaphoreType.DMA,
            pltpu.SemaphoreType.DMA,
        ],
    )


def _mm_first(x, w, sw, *, bm, dh):
    n, d = x.shape

    def body(x_ref, w_ref, sw_ref, g_ref, gi_ref):
        g = jnp.dot(x_ref[...], w_ref[...], preferred_element_type=jnp.float32)
        gi = g * sw_ref[...]
        g_ref[0] = g[:, :dh]
        g_ref[1] = g[:, dh:]
        gi_ref[0] = gi[:, :dh]
        gi_ref[1] = gi[:, dh:]

    return pl.pallas_call(
        body,
        grid=(n // bm,),
        in_specs=[
            pl.BlockSpec((bm, d), lambda i: (i, 0)),
            pl.BlockSpec((d, d), lambda i: (0, 0)),
            pl.BlockSpec((bm, 1), lambda i: (i, 0)),
        ],
        out_specs=[
            pl.BlockSpec((NC, bm, dh), lambda i: (0, i, 0)),
            pl.BlockSpec((NC, bm, dh), lambda i: (0, i, 0)),
        ],
        out_shape=[
            jax.ShapeDtypeStruct((NC, n, dh), jnp.float32),
            jax.ShapeDtypeStruct((NC, n, dh), jnp.float32),
        ],
    )(x, w, sw)


def _mm_mid(agg, b_prev, w, sw, *, bm, dh):
    nc, n, _ = agg.shape
    d = w.shape[0]

    def body(a_ref, b_ref, w_ref, sw_ref, g_ref, gi_ref):
        h = jnp.concatenate([a_ref[0], a_ref[1]], axis=1) + b_ref[...]
        h = jnp.maximum(h, 0.0)
        g = jnp.dot(h, w_ref[...], preferred_element_type=jnp.float32)
        gi = g * sw_ref[...]
        g_ref[0] = g[:, :dh]
        g_ref[1] = g[:, dh:]
        gi_ref[0] = gi[:, :dh]
        gi_ref[1] = gi[:, dh:]

    return pl.pallas_call(
        body,
        grid=(n // bm,),
        in_specs=[
            pl.BlockSpec((NC, bm, dh), lambda i: (0, i, 0)),
            pl.BlockSpec((1, d), lambda i: (0, 0)),
            pl.BlockSpec((d, d), lambda i: (0, 0)),
            pl.BlockSpec((bm, 1), lambda i: (i, 0)),
        ],
        out_specs=[
            pl.BlockSpec((NC, bm, dh), lambda i: (0, i, 0)),
            pl.BlockSpec((NC, bm, dh), lambda i: (0, i, 0)),
        ],
        out_shape=[
            jax.ShapeDtypeStruct((NC, n, dh), jnp.float32),
            jax.ShapeDtypeStruct((NC, n, dh), jnp.float32),
        ],
    )(agg, b_prev, w, sw)


def _mm_last(agg, b_prev, w, b_out, *, bm, dh):
    nc, n, _ = agg.shape
    d = w.shape[0]

    def body(a_ref, b_ref, w_ref, bo_ref, o_ref):
        h = jnp.concatenate([a_ref[0], a_ref[1]], axis=1) + b_ref[...]
        h = jnp.maximum(h, 0.0)
        o_ref[...] = (jnp.dot(h, w_ref[...], preferred_element_type=jnp.float32)
                      + bo_ref[...])

    return pl.pallas_call(
        body,
        grid=(n // bm,),
        in_specs=[
            pl.BlockSpec((NC, bm, dh), lambda i: (0, i, 0)),
            pl.BlockSpec((1, d), lambda i: (0, 0)),
            pl.BlockSpec((d, d), lambda i: (0, 0)),
            pl.BlockSpec((1, d), lambda i: (0, 0)),
        ],
        out_specs=pl.BlockSpec((bm, d), lambda i: (i, 0)),
        out_shape=jax.ShapeDtypeStruct((n, d), jnp.float32),
    )(agg, b_prev, w, b_out)


def kernel(x, edge_index, edge_attr, W1, b1, W2, b2, W3, b3, Wout, bout):
    n, d = x.shape
    e = edge_index.shape[1]
    dh = d // 2
    cw = 64
    kch = -(-e // (NW * CH))
    ep = NW * CH * kch
    nre = ep // CH
    nr2 = ep // cw
    npad = -(-n // 512) * 512
    bm = 512

    pad = ep - e
    src = jnp.concatenate([edge_index[0], jnp.zeros((pad,), jnp.int32)])
    dst = jnp.concatenate([edge_index[1], jnp.zeros((pad,), jnp.int32)])
    wp = jnp.concatenate([edge_attr, jnp.zeros((pad,), edge_attr.dtype)])
    src2 = jnp.stack([src, src + npad]).reshape(NC * nr2, cw)
    srcn = src.reshape(nre, CH)
    dstn = dst.reshape(nre, CH)
    dst2 = dst.reshape(nr2, cw)
    wr = wp.reshape(nre, CH)
    xp = jnp.pad(x, ((0, npad - n), (0, 0)))

    layer = _make_layer(npad, nr2, cw, dh)

    degp = _make_deg(npad, nre, kch)(dstn, wr)
    dis, selfw = _dis_tc(degp.reshape(NC, npad), npad)
    norm_r = _make_norm(npad, nre, kch)(srcn, dstn, wr, dis.reshape(npad))
    sw2 = selfw.reshape(npad, 1)

    g, gi = _mm_first(xp, W1, sw2, bm=bm, dh=dh)
    agg = layer(g.reshape(NC * npad, dh), gi.reshape(NC * npad, dh),
                src2, dst2, norm_r).reshape(NC, npad, dh)
    g, gi = _mm_mid(agg, b1.reshape(1, d), W2, sw2, bm=bm, dh=dh)
    agg = layer(g.reshape(NC * npad, dh), gi.reshape(NC * npad, dh),
                src2, dst2, norm_r).reshape(NC, npad, dh)
    g, gi = _mm_mid(agg, b2.reshape(1, d), W3, sw2, bm=bm, dh=dh)
    agg = layer(g.reshape(NC * npad, dh), gi.reshape(NC * npad, dh),
                src2, dst2, norm_r).reshape(NC, npad, dh)
    out = _mm_last(agg, b3.reshape(1, d), Wout, bout.reshape(1, d),
                   bm=bm, dh=dh)
    return out[:n]
